# Initial kernel scaffold; baseline (speedup 1.0000x reference)
import functools

import jax
import jax.numpy as jnp
from jax import lax
from jax.experimental import pallas as pl
from jax.experimental.pallas import tpu as pltpu
from jax.experimental.pallas import tpu_sc as plsc

_NUM_CORES = 2
_NUM_SUBCORES = 16
_NUM_WORKERS = _NUM_CORES * _NUM_SUBCORES
_LANES = 16


def _rne_bf16_bits(x_f32):
    xi = lax.bitcast_convert_type(x_f32, jnp.int32)
    rounded = xi + jnp.int32(0x7FFF) + ((xi >> 16) & jnp.int32(1))
    return (rounded >> 16) & jnp.int32(0xFFFF)


def _project_body(table_ref, wt_ref, out_ref):
    p = jnp.dot(table_ref[...], wt_ref[...], preferred_element_type=jnp.float32)
    c0 = _rne_bf16_bits(p[:, 0:1])
    c1 = _rne_bf16_bits(p[:, 1:2])
    out_ref[...] = c0 | (c1 << 16)


def _project_pack(table, W, inv_l):
    v, d = table.shape
    block = 10000
    grid = v // block
    wt = jnp.zeros((d, 8), jnp.float32).at[:, :2].set(W.T * inv_l)
    return pl.pallas_call(
        _project_body,
        grid=(grid,),
        in_specs=[
            pl.BlockSpec((block, d), lambda i: (i, 0)),
            pl.BlockSpec((d, 8), lambda i: (0, 0)),
        ],
        out_specs=pl.BlockSpec((block, 1), lambda i: (i, 0)),
        out_shape=jax.ShapeDtypeStruct((v, 1), jnp.int32),
    )(table, wt)


def _make_bag_kernel(v, b_total, hist, rows_per_chunk, n_chunks):
    rows_per_worker = rows_per_chunk * n_chunks
    groups_per_chunk = rows_per_chunk // _LANES
    chunk_words = rows_per_chunk * hist
    mesh = plsc.VectorSubcoreMesh(core_axis_name="c", subcore_axis_name="s")

    @functools.partial(
        pl.kernel,
        out_type=jax.ShapeDtypeStruct((2 * b_total,), jnp.float32),
        mesh=mesh,
        scratch_types=[
            pltpu.VMEM((v,), jnp.int32),
            pltpu.VMEM((2, chunk_words), jnp.int32),
            pltpu.VMEM((2, rows_per_worker), jnp.float32),
            pltpu.VMEM((16,), jnp.float32),
            pltpu.SemaphoreType.DMA,
            pltpu.SemaphoreType.DMA,
            pltpu.SemaphoreType.DMA,
        ],
    )
    def bag(idx_hbm, tbl_hbm, bias_hbm, out_hbm, tbl_v, idx_v, out_v, bias_v,
            sem_a, sem_b, sem_t):
        wid = lax.axis_index("c") * _NUM_SUBCORES + lax.axis_index("s")
        row0 = wid * rows_per_worker

        pltpu.sync_copy(bias_hbm, bias_v)
        b0 = bias_v[0]
        b1 = bias_v[1]

        tbl_copy = pltpu.make_async_copy(tbl_hbm, tbl_v, sem_t)
        tbl_copy.start()

        sems = [sem_a, sem_b]

        def idx_copy(chunk):
            return pltpu.make_async_copy(
                idx_hbm.at[pl.ds((row0 + chunk * rows_per_chunk) * hist,
                                 chunk_words)],
                idx_v.at[chunk % 2],
                sems[chunk % 2],
            )

        idx_copy(0).start()
        tbl_copy.wait()

        lane = lax.iota(jnp.int32, _LANES)
        zero16 = jnp.zeros((_LANES,), jnp.float32)
        bias0 = zero16 + b0
        bias1 = zero16 + b1

        for ci in range(n_chunks):
            slot = ci % 2
            idx_copy(ci).wait()
            if ci + 1 < n_chunks:
                idx_copy(ci + 1).start()
            slot_v = jnp.zeros((_LANES,), jnp.int32) + slot

            def process_group(gi, _, _slot_v=slot_v, _ci=ci):
                pos0 = (gi * _LANES + lane) * hist

                def step(tt, accs):
                    a0, a1, a2, a3 = accs
                    p = pos0 + 2 * tt
                    i0 = plsc.load_gather(idx_v, [_slot_v, p])
                    i1 = plsc.load_gather(idx_v, [_slot_v, p + 1])
                    e0 = plsc.load_gather(tbl_v, [i0])
                    e1 = plsc.load_gather(tbl_v, [i1])
                    a0 = a0 + plsc.bitcast(e0 << 16, jnp.float32)
                    a1 = a1 + plsc.bitcast(e0 & jnp.int32(-65536), jnp.float32)
                    a2 = a2 + plsc.bitcast(e1 << 16, jnp.float32)
                    a3 = a3 + plsc.bitcast(e1 & jnp.int32(-65536), jnp.float32)
                    return a0, a1, a2, a3

                a0, a1, a2, a3 = lax.fori_loop(
                    0, hist // 2, step, (bias0, bias1, zero16, zero16))
                out_row = _ci * rows_per_chunk + gi * _LANES
                out_v[0, pl.ds(out_row, _LANES)] = a0 + a2
                out_v[1, pl.ds(out_row, _LANES)] = a1 + a3
                return 0

            lax.fori_loop(0, groups_per_chunk, process_group, 0)

        out_copy0 = pltpu.make_async_copy(
            out_v.at[0], out_hbm.at[pl.ds(row0, rows_per_worker)], sem_t)
        out_copy1 = pltpu.make_async_copy(
            out_v.at[1], out_hbm.at[pl.ds(b_total + row0, rows_per_worker)],
            sem_t)
        out_copy0.start()
        out_copy1.start()
        out_copy0.wait()
        out_copy1.wait()

    return bag


def kernel(token_index, table, W, b):
    b_total, hist = token_index.shape
    v, _ = table.shape
    packed = _project_pack(table, W, 1.0 / hist).reshape(v)
    rows_per_worker = b_total // _NUM_WORKERS
    n_chunks = 8
    rows_per_chunk = rows_per_worker // n_chunks
    bias16 = jnp.zeros((16,), jnp.float32).at[:2].set(b)
    bag = _make_bag_kernel(v, b_total, hist, rows_per_chunk, n_chunks)
    flat = bag(token_index.reshape(-1), packed, bias16)
    return flat.reshape(2, b_total).T


# speedup vs baseline: 79.5877x; 79.5877x over previous
import functools

import jax
import jax.numpy as jnp
from jax import lax
from jax.experimental import pallas as pl
from jax.experimental.pallas import tpu as pltpu
from jax.experimental.pallas import tpu_sc as plsc

_NUM_CORES = 2
_NUM_SUBCORES = 16
_NUM_WORKERS = _NUM_CORES * _NUM_SUBCORES
_LANES = 16


def _rne_bf16_bits(x_f32):
    xi = lax.bitcast_convert_type(x_f32, jnp.int32)
    rounded = xi + jnp.int32(0x7FFF) + ((xi >> 16) & jnp.int32(1))
    return (rounded >> 16) & jnp.int32(0xFFFF)


def _project_body(table_ref, wt_ref, out_ref):
    p = jnp.dot(table_ref[...], wt_ref[...], preferred_element_type=jnp.float32)
    c0 = _rne_bf16_bits(p[:, 0:1])
    c1 = _rne_bf16_bits(p[:, 1:2])
    out_ref[...] = c0 | (c1 << 16)


def _project_pack(table, W, inv_l):
    v, d = table.shape
    block = 10000
    grid = v // block
    wt = jnp.zeros((d, 8), jnp.float32).at[:, :2].set(W.T * inv_l)
    return pl.pallas_call(
        _project_body,
        grid=(grid,),
        in_specs=[
            pl.BlockSpec((block, d), lambda i: (i, 0)),
            pl.BlockSpec((d, 8), lambda i: (0, 0)),
        ],
        out_specs=pl.BlockSpec((block, 1), lambda i: (i, 0)),
        out_shape=jax.ShapeDtypeStruct((v, 1), jnp.int32),
    )(table, wt)


def _make_bag_kernel(v, b_total, hist, rows_per_chunk, n_chunks):
    rows_per_worker = rows_per_chunk * n_chunks
    groups_per_chunk = rows_per_chunk // _LANES
    chunk_words = rows_per_chunk * hist
    mesh = plsc.VectorSubcoreMesh(core_axis_name="c", subcore_axis_name="s")

    @functools.partial(
        pl.kernel,
        out_type=jax.ShapeDtypeStruct((2 * b_total,), jnp.float32),
        mesh=mesh,
        scratch_types=[
            pltpu.VMEM((v,), jnp.int32),
            pltpu.VMEM((chunk_words,), jnp.int32),
            pltpu.VMEM((chunk_words,), jnp.int32),
            pltpu.VMEM((2, rows_per_worker), jnp.float32),
            pltpu.VMEM((2, 16), jnp.float32),
            pltpu.SemaphoreType.DMA,
            pltpu.SemaphoreType.DMA,
            pltpu.SemaphoreType.DMA,
        ],
        compiler_params=pltpu.CompilerParams(needs_layout_passes=False),
    )
    def bag(idx_hbm, tbl_hbm, bias_hbm, out_hbm, tbl_v, idx_a, idx_b, out_v,
            bias_v, sem_a, sem_b, sem_t):
        wid = lax.axis_index("c") * _NUM_SUBCORES + lax.axis_index("s")
        row0 = wid * rows_per_worker

        pltpu.sync_copy(bias_hbm, bias_v)

        tbl_copy = pltpu.make_async_copy(tbl_hbm, tbl_v, sem_t)
        tbl_copy.start()

        sems = [sem_a, sem_b]
        bufs = [idx_a, idx_b]

        def idx_copy(chunk):
            return pltpu.make_async_copy(
                idx_hbm.at[pl.ds((row0 + chunk * rows_per_chunk) * hist,
                                 chunk_words)],
                bufs[chunk % 2],
                sems[chunk % 2],
            )

        idx_copy(0).start()
        tbl_copy.wait()

        lane = lax.iota(jnp.int32, _LANES)
        zero16 = jnp.zeros((_LANES,), jnp.float32)
        bias0 = bias_v[0, :]
        bias1 = bias_v[1, :]

        for ci in range(n_chunks):
            idx_copy(ci).wait()
            if ci + 1 < n_chunks:
                idx_copy(ci + 1).start()
            buf = bufs[ci % 2]

            def process_group(gi, _, _buf=buf, _ci=ci):
                pos0 = (gi * _LANES + lane) * hist

                def step(tt, accs):
                    a0, a1, a2, a3 = accs
                    p = pos0 + 2 * tt
                    i0 = plsc.load_gather(_buf, [p])
                    i1 = plsc.load_gather(_buf, [p + 1])
                    e0 = plsc.load_gather(tbl_v, [i0])
                    e1 = plsc.load_gather(tbl_v, [i1])
                    a0 = a0 + plsc.bitcast(e0 << 16, jnp.float32)
                    a1 = a1 + plsc.bitcast(e0 & jnp.int32(-65536), jnp.float32)
                    a2 = a2 + plsc.bitcast(e1 << 16, jnp.float32)
                    a3 = a3 + plsc.bitcast(e1 & jnp.int32(-65536), jnp.float32)
                    return a0, a1, a2, a3

                a0, a1, a2, a3 = lax.fori_loop(
                    0, hist // 2, step, (bias0, bias1, zero16, zero16))
                out_row = _ci * rows_per_chunk + gi * _LANES
                out_v[0, pl.ds(out_row, _LANES)] = a0 + a2
                out_v[1, pl.ds(out_row, _LANES)] = a1 + a3
                return 0

            lax.fori_loop(0, groups_per_chunk, process_group, 0)

        out_copy0 = pltpu.make_async_copy(
            out_v.at[0], out_hbm.at[pl.ds(row0, rows_per_worker)], sem_t)
        out_copy1 = pltpu.make_async_copy(
            out_v.at[1], out_hbm.at[pl.ds(b_total + row0, rows_per_worker)],
            sem_t)
        out_copy0.start()
        out_copy1.start()
        out_copy0.wait()
        out_copy1.wait()

    return bag


def kernel(token_index, table, W, b):
    b_total, hist = token_index.shape
    v, _ = table.shape
    packed = _project_pack(table, W, 1.0 / hist).reshape(v)
    rows_per_worker = b_total // _NUM_WORKERS
    n_chunks = 8
    rows_per_chunk = rows_per_worker // n_chunks
    bias16 = jnp.broadcast_to(b[:, None], (2, 16)).astype(jnp.float32)
    bag = _make_bag_kernel(v, b_total, hist, rows_per_chunk, n_chunks)
    flat = bag(token_index.reshape(-1), packed, bias16)
    return flat.reshape(2, b_total).T
